# baseline (speedup 1.0000x reference)
import functools

import jax
import jax.numpy as jnp
from jax import lax
from jax.experimental import pallas as pl
from jax.experimental.pallas import tpu as pltpu
from jax.experimental.pallas import tpu_sc as plsc

_N = 10000
_NP = 10240
_E = 320000
_EP = 327680
_G = 16
_DIN = 128
_DH = 128

_NTILES = 32
_EPT = _EP // _NTILES
_K = 128
_NCH = _EPT // _K
_RPT = _NP // 16

_mesh = plsc.VectorSubcoreMesh(core_axis_name="c", subcore_axis_name="s")


def _deg_body(dst_hbm, ones_hbm, zeros_hbm, out_hbm, dstl, ones_v, acc, sem):
    c = lax.axis_index("c")
    s = lax.axis_index("s")
    row0 = s * _RPT
    crow = (c * 16 + s) * _NCH
    pltpu.sync_copy(zeros_hbm, acc.at[pl.ds(row0, _RPT)])
    pltpu.sync_copy(ones_hbm, ones_v)
    pltpu.sync_copy(dst_hbm.at[pl.ds(crow, _NCH)], dstl)
    plsc.subcore_barrier()

    def body(i, carry):
        pltpu.sync_copy(ones_v, acc.at[dstl.at[i]], add=True)
        return carry

    lax.fori_loop(0, _NCH, body, 0)
    plsc.subcore_barrier()
    pltpu.sync_copy(acc.at[pl.ds(row0, _RPT)], out_hbm.at[c, pl.ds(row0, _RPT)])


_sc_deg = pl.kernel(
    _deg_body,
    out_type=jax.ShapeDtypeStruct((2, _NP), jnp.float32),
    mesh=_mesh,
    scratch_types=[
        pltpu.VMEM((_NCH, _K), jnp.int32),
        pltpu.VMEM((_K,), jnp.float32),
        pltpu.VMEM_SHARED((_NP,), jnp.float32),
        pltpu.SemaphoreType.DMA,
    ],
)


def _make_spmm(width):
    def body(v_hbm, src_hbm, dst_hbm, zeros_hbm, out_hbm,
             srcl, dstl, rows0, rows1, acc, sem0, sem1, sems0, sems1, semz):
        c = lax.axis_index("c")
        s = lax.axis_index("s")
        row0 = s * _RPT
        crow = (c * 16 + s) * _NCH
        pltpu.async_copy(zeros_hbm, acc.at[pl.ds(row0, _RPT)], semz)

        nch_h = _NCH // 2
        npairs = nch_h // 2
        for h in range(2):
            pltpu.sync_copy(src_hbm.at[pl.ds(crow + h * nch_h, nch_h)], srcl)
            pltpu.sync_copy(dst_hbm.at[pl.ds(crow + h * nch_h, nch_h)], dstl)
            pltpu.async_copy(v_hbm.at[srcl.at[0]], rows0, sem0)
            pltpu.async_copy(v_hbm.at[srcl.at[1]], rows1, sem1)
            if h == 0:
                pltpu.make_async_copy(zeros_hbm, acc.at[pl.ds(row0, _RPT)],
                                      semz).wait()
                plsc.subcore_barrier()

            def step(j, carry):
                i0 = 2 * j
                pltpu.make_async_copy(v_hbm.at[srcl.at[i0]], rows0,
                                      sem0).wait()
                pltpu.async_copy(rows0, acc.at[dstl.at[i0]], sems0, add=True)
                pltpu.make_async_copy(v_hbm.at[srcl.at[i0 + 1]], rows1,
                                      sem1).wait()
                pltpu.async_copy(rows1, acc.at[dstl.at[i0 + 1]], sems1,
                                 add=True)
                pltpu.make_async_copy(rows0, acc.at[dstl.at[i0]],
                                      sems0).wait()

                @pl.when(j < npairs - 1)
                def _():
                    pltpu.async_copy(v_hbm.at[srcl.at[i0 + 2]], rows0, sem0)

                pltpu.make_async_copy(rows1, acc.at[dstl.at[i0 + 1]],
                                      sems1).wait()

                @pl.when(j < npairs - 1)
                def _():
                    pltpu.async_copy(v_hbm.at[srcl.at[i0 + 3]], rows1, sem1)

                return carry

            lax.fori_loop(0, npairs, step, 0)
        plsc.subcore_barrier()
        pltpu.sync_copy(acc.at[pl.ds(row0, _RPT)],
                        out_hbm.at[c, pl.ds(row0, _RPT)])

    return pl.kernel(
        body,
        out_type=jax.ShapeDtypeStruct((2, _NP, width), jnp.float32),
        mesh=_mesh,
        scratch_types=[
            pltpu.VMEM((_NCH // 2, _K), jnp.int32),
            pltpu.VMEM((_NCH // 2, _K), jnp.int32),
            pltpu.VMEM((_K, width), jnp.float32),
            pltpu.VMEM((_K, width), jnp.float32),
            pltpu.VMEM_SHARED((_NP, width), jnp.float32),
            pltpu.SemaphoreType.DMA,
            pltpu.SemaphoreType.DMA,
            pltpu.SemaphoreType.DMA,
            pltpu.SemaphoreType.DMA,
            pltpu.SemaphoreType.DMA,
        ],
    )


_spmm128 = _make_spmm(128)


_B = 1024
_GRID = _NP // _B


def _tc0_body(degp_ref, x_ref, w_ref, dinv_ref, out_ref):
    deg = (degp_ref[0, :] + degp_ref[1, :] + 1.0).reshape(_B, 1)
    dinv = lax.rsqrt(deg)
    dinv_ref[...] = dinv
    out_ref[...] = dinv * jnp.dot(x_ref[...], w_ref[...],
                                  preferred_element_type=jnp.float32)


def _tc0(degp, xp, w1):
    return pl.pallas_call(
        _tc0_body,
        grid=(_GRID,),
        in_specs=[
            pl.BlockSpec((2, _B), lambda i: (0, i)),
            pl.BlockSpec((_B, _DIN), lambda i: (i, 0)),
            pl.BlockSpec((_DIN, _DH), lambda i: (0, 0)),
        ],
        out_specs=[
            pl.BlockSpec((_B, 1), lambda i: (i, 0)),
            pl.BlockSpec((_B, _DH), lambda i: (i, 0)),
        ],
        out_shape=[
            jax.ShapeDtypeStruct((_NP, 1), jnp.float32),
            jax.ShapeDtypeStruct((_NP, _DH), jnp.float32),
        ],
    )(degp, xp, w1)


def _tc_mid_body(dinv_ref, p_ref, v_ref, b_ref, w_ref, out_ref):
    dinv = dinv_ref[...]
    t = p_ref[0] + p_ref[1] + v_ref[...]
    h = jnp.maximum(dinv * t + b_ref[...], 0.0)
    out_ref[...] = dinv * jnp.dot(h, w_ref[...],
                                  preferred_element_type=jnp.float32)


def _tc_mid(dinv, p, v, b, w, win, wout):
    return pl.pallas_call(
        _tc_mid_body,
        grid=(_GRID,),
        in_specs=[
            pl.BlockSpec((_B, 1), lambda i: (i, 0)),
            pl.BlockSpec((2, _B, win), lambda i: (0, i, 0)),
            pl.BlockSpec((_B, win), lambda i: (i, 0)),
            pl.BlockSpec((1, win), lambda i: (0, 0)),
            pl.BlockSpec((win, wout), lambda i: (0, 0)),
        ],
        out_specs=pl.BlockSpec((_B, wout), lambda i: (i, 0)),
        out_shape=jax.ShapeDtypeStruct((_NP, wout), jnp.float32),
    )(dinv, p, v, b, w)


def _tc_final_body(dinv_ref, p_ref, v_ref, b_ref, bat_ref, wfc_ref, bfc_ref,
                   out_ref, sums, counts):
    i = pl.program_id(0)
    dinv = dinv_ref[...]
    t = p_ref[0] + p_ref[1] + v_ref[...]
    h = jnp.maximum(dinv * t + b_ref[...], 0.0)
    ids = bat_ref[...]
    groups = lax.broadcasted_iota(jnp.int32, (_B, _G), 1)
    mask = (ids == groups).astype(jnp.float32)
    blk_sums = lax.dot_general(mask, h, (((0,), (0,)), ((), ())),
                               preferred_element_type=jnp.float32)
    blk_counts = lax.dot_general(mask, jnp.ones((_B, 1), jnp.float32),
                                 (((0,), (0,)), ((), ())),
                                 preferred_element_type=jnp.float32)

    @pl.when(i == 0)
    def _():
        sums[...] = jnp.zeros_like(sums)
        counts[...] = jnp.zeros_like(counts)

    sums[...] += blk_sums
    counts[...] += blk_counts

    @pl.when(i == _GRID - 1)
    def _():
        pooled = sums[...] / jnp.maximum(counts[...], 1.0)
        out_ref[...] = (jnp.dot(pooled, wfc_ref[...],
                                preferred_element_type=jnp.float32)
                        + bfc_ref[...])


def _tc_final(dinv, p, v, b, batp, wfc, bfc):
    return pl.pallas_call(
        _tc_final_body,
        grid=(_GRID,),
        in_specs=[
            pl.BlockSpec((_B, 1), lambda i: (i, 0)),
            pl.BlockSpec((2, _B, 128), lambda i: (0, i, 0)),
            pl.BlockSpec((_B, 128), lambda i: (i, 0)),
            pl.BlockSpec((1, 128), lambda i: (0, 0)),
            pl.BlockSpec((_B, 1), lambda i: (i, 0)),
            pl.BlockSpec((128, 1), lambda i: (0, 0)),
            pl.BlockSpec((1, 1), lambda i: (0, 0)),
        ],
        out_specs=pl.BlockSpec((_G, 1), lambda i: (0, 0)),
        out_shape=jax.ShapeDtypeStruct((_G, 1), jnp.float32),
        scratch_shapes=[
            pltpu.VMEM((_G, 128), jnp.float32),
            pltpu.VMEM((_G, 1), jnp.float32),
        ],
    )(dinv, p, v, b, batp, wfc, bfc)


def kernel(x, edge_index, batch, W1, b1, W2, b2, W3, b3, Wfc, bfc):
    npad = _NP - _N
    dum = (jnp.arange(_EP - _E, dtype=jnp.int32) % npad) + _N
    srcp = jnp.concatenate([edge_index[0], dum]).reshape(_EP // _K, _K)
    dstp = jnp.concatenate([edge_index[1], dum]).reshape(_EP // _K, _K)
    xp = jnp.concatenate([x, jnp.zeros((npad, _DIN), jnp.float32)])
    batp = jnp.concatenate(
        [batch, jnp.full((npad,), _G, batch.dtype)]).reshape(_NP, 1)

    ones1 = jnp.ones((_K,), jnp.float32)
    z1 = jnp.zeros((_RPT,), jnp.float32)
    z128 = jnp.zeros((_RPT, 128), jnp.float32)

    W3p = jnp.concatenate([W3, jnp.zeros((_DH, 64), jnp.float32)], axis=1)
    b3p = jnp.concatenate([b3, jnp.zeros((64,), jnp.float32)]).reshape(1, -1)
    Wfcp = jnp.concatenate([Wfc, jnp.zeros((64, 1), jnp.float32)], axis=0)

    degp = _sc_deg(dstp, ones1, z1)
    dinv, v1 = _tc0(degp, xp, W1)
    p1 = _spmm128(v1, srcp, dstp, z128)
    v2 = _tc_mid(dinv, p1, v1, b1.reshape(1, -1), W2, 128, 128)
    p2 = _spmm128(v2, srcp, dstp, z128)
    v3 = _tc_mid(dinv, p2, v2, b2.reshape(1, -1), W3p, 128, 128)
    p3 = _spmm128(v3, srcp, dstp, z128)
    out = _tc_final(dinv, p3, v3, b3p, batp, Wfcp, bfc.reshape(1, -1))
    return out.reshape(_G)


# speedup vs baseline: 1.2576x; 1.2576x over previous
import functools

import jax
import jax.numpy as jnp
from jax import lax
from jax.experimental import pallas as pl
from jax.experimental.pallas import tpu as pltpu
from jax.experimental.pallas import tpu_sc as plsc

_N = 10000
_NP = 10240
_E = 320000
_EP = 327680
_G = 16
_DIN = 128
_DH = 128

_NTILES = 32
_EPT = _EP // _NTILES
_K = 128
_NCH = _EPT // _K
_RPT = _NP // 16

_mesh = plsc.VectorSubcoreMesh(core_axis_name="c", subcore_axis_name="s")


def _deg_body(dst_hbm, ones_hbm, zeros_hbm, out_hbm, dstl, ones_v, acc, sem):
    c = lax.axis_index("c")
    s = lax.axis_index("s")
    row0 = s * _RPT
    crow = (c * 16 + s) * _NCH
    pltpu.sync_copy(zeros_hbm, acc.at[pl.ds(row0, _RPT)])
    pltpu.sync_copy(ones_hbm, ones_v)
    pltpu.sync_copy(dst_hbm.at[pl.ds(crow, _NCH)], dstl)
    plsc.subcore_barrier()

    def body(i, carry):
        pltpu.sync_copy(ones_v, acc.at[dstl.at[i]], add=True)
        return carry

    lax.fori_loop(0, _NCH, body, 0)
    plsc.subcore_barrier()
    pltpu.sync_copy(acc.at[pl.ds(row0, _RPT)], out_hbm.at[c, pl.ds(row0, _RPT)])


_sc_deg = pl.kernel(
    _deg_body,
    out_type=jax.ShapeDtypeStruct((2, _NP), jnp.float32),
    mesh=_mesh,
    scratch_types=[
        pltpu.VMEM((_NCH, _K), jnp.int32),
        pltpu.VMEM((_K,), jnp.float32),
        pltpu.VMEM_SHARED((_NP,), jnp.float32),
        pltpu.SemaphoreType.DMA,
    ],
)


def _make_spmm(width):
    def body(v_hbm, src_hbm, dst_hbm, zeros_hbm, out_hbm,
             srcl, dstl, rows0, rows1, acc, sem0, sem1, semz):
        c = lax.axis_index("c")
        s = lax.axis_index("s")
        row0 = s * _RPT
        crow = (c * 16 + s) * _NCH
        pltpu.async_copy(zeros_hbm, acc.at[pl.ds(row0, _RPT)], semz)

        nch_h = _NCH // 2
        npairs = nch_h // 2
        for h in range(2):
            pltpu.sync_copy(src_hbm.at[pl.ds(crow + h * nch_h, nch_h)], srcl)
            pltpu.sync_copy(dst_hbm.at[pl.ds(crow + h * nch_h, nch_h)], dstl)
            pltpu.async_copy(v_hbm.at[srcl.at[0]], rows0, sem0)
            if h == 0:
                pltpu.make_async_copy(zeros_hbm, acc.at[pl.ds(row0, _RPT)],
                                      semz).wait()
                plsc.subcore_barrier()

            def step(j, carry):
                i0 = 2 * j
                pltpu.async_copy(v_hbm.at[srcl.at[i0 + 1]], rows1, sem1)
                pltpu.make_async_copy(v_hbm.at[srcl.at[i0]], rows0,
                                      sem0).wait()
                pltpu.sync_copy(rows0, acc.at[dstl.at[i0]], add=True)

                @pl.when(j < npairs - 1)
                def _():
                    pltpu.async_copy(v_hbm.at[srcl.at[i0 + 2]], rows0, sem0)

                pltpu.make_async_copy(v_hbm.at[srcl.at[i0 + 1]], rows1,
                                      sem1).wait()
                pltpu.sync_copy(rows1, acc.at[dstl.at[i0 + 1]], add=True)
                return carry

            lax.fori_loop(0, npairs, step, 0)
        plsc.subcore_barrier()
        pltpu.sync_copy(acc.at[pl.ds(row0, _RPT)],
                        out_hbm.at[c, pl.ds(row0, _RPT)])

    return pl.kernel(
        body,
        out_type=jax.ShapeDtypeStruct((2, _NP, width), jnp.float32),
        mesh=_mesh,
        scratch_types=[
            pltpu.VMEM((_NCH // 2, _K), jnp.int32),
            pltpu.VMEM((_NCH // 2, _K), jnp.int32),
            pltpu.VMEM((_K, width), jnp.float32),
            pltpu.VMEM((_K, width), jnp.float32),
            pltpu.VMEM_SHARED((_NP, width), jnp.float32),
            pltpu.SemaphoreType.DMA,
            pltpu.SemaphoreType.DMA,
            pltpu.SemaphoreType.DMA,
        ],
    )


_spmm128 = _make_spmm(128)


_B = 1024
_GRID = _NP // _B


def _tc0_body(degp_ref, x_ref, w_ref, dinv_ref, out_ref):
    deg = (degp_ref[0, :] + degp_ref[1, :] + 1.0).reshape(_B, 1)
    dinv = lax.rsqrt(deg)
    dinv_ref[...] = dinv
    out_ref[...] = dinv * jnp.dot(x_ref[...], w_ref[...],
                                  preferred_element_type=jnp.float32)


def _tc0(degp, xp, w1):
    return pl.pallas_call(
        _tc0_body,
        grid=(_GRID,),
        in_specs=[
            pl.BlockSpec((2, _B), lambda i: (0, i)),
            pl.BlockSpec((_B, _DIN), lambda i: (i, 0)),
            pl.BlockSpec((_DIN, _DH), lambda i: (0, 0)),
        ],
        out_specs=[
            pl.BlockSpec((_B, 1), lambda i: (i, 0)),
            pl.BlockSpec((_B, _DH), lambda i: (i, 0)),
        ],
        out_shape=[
            jax.ShapeDtypeStruct((_NP, 1), jnp.float32),
            jax.ShapeDtypeStruct((_NP, _DH), jnp.float32),
        ],
    )(degp, xp, w1)


def _tc_mid_body(dinv_ref, p_ref, v_ref, b_ref, w_ref, out_ref):
    dinv = dinv_ref[...]
    t = p_ref[0] + p_ref[1] + v_ref[...]
    h = jnp.maximum(dinv * t + b_ref[...], 0.0)
    out_ref[...] = dinv * jnp.dot(h, w_ref[...],
                                  preferred_element_type=jnp.float32)


def _tc_mid(dinv, p, v, b, w, win, wout):
    return pl.pallas_call(
        _tc_mid_body,
        grid=(_GRID,),
        in_specs=[
            pl.BlockSpec((_B, 1), lambda i: (i, 0)),
            pl.BlockSpec((2, _B, win), lambda i: (0, i, 0)),
            pl.BlockSpec((_B, win), lambda i: (i, 0)),
            pl.BlockSpec((1, win), lambda i: (0, 0)),
            pl.BlockSpec((win, wout), lambda i: (0, 0)),
        ],
        out_specs=pl.BlockSpec((_B, wout), lambda i: (i, 0)),
        out_shape=jax.ShapeDtypeStruct((_NP, wout), jnp.float32),
    )(dinv, p, v, b, w)


def _tc_final_body(dinv_ref, p_ref, v_ref, b_ref, bat_ref, wfc_ref, bfc_ref,
                   out_ref, sums, counts):
    i = pl.program_id(0)
    dinv = dinv_ref[...]
    t = p_ref[0] + p_ref[1] + v_ref[...]
    h = jnp.maximum(dinv * t + b_ref[...], 0.0)
    ids = bat_ref[...]
    groups = lax.broadcasted_iota(jnp.int32, (_B, _G), 1)
    mask = (ids == groups).astype(jnp.float32)
    blk_sums = lax.dot_general(mask, h, (((0,), (0,)), ((), ())),
                               preferred_element_type=jnp.float32)
    blk_counts = lax.dot_general(mask, jnp.ones((_B, 1), jnp.float32),
                                 (((0,), (0,)), ((), ())),
                                 preferred_element_type=jnp.float32)

    @pl.when(i == 0)
    def _():
        sums[...] = jnp.zeros_like(sums)
        counts[...] = jnp.zeros_like(counts)

    sums[...] += blk_sums
    counts[...] += blk_counts

    @pl.when(i == _GRID - 1)
    def _():
        pooled = sums[...] / jnp.maximum(counts[...], 1.0)
        out_ref[...] = (jnp.dot(pooled, wfc_ref[...],
                                preferred_element_type=jnp.float32)
                        + bfc_ref[...])


def _tc_final(dinv, p, v, b, batp, wfc, bfc):
    return pl.pallas_call(
        _tc_final_body,
        grid=(_GRID,),
        in_specs=[
            pl.BlockSpec((_B, 1), lambda i: (i, 0)),
            pl.BlockSpec((2, _B, 128), lambda i: (0, i, 0)),
            pl.BlockSpec((_B, 128), lambda i: (i, 0)),
            pl.BlockSpec((1, 128), lambda i: (0, 0)),
            pl.BlockSpec((_B, 1), lambda i: (i, 0)),
            pl.BlockSpec((128, 1), lambda i: (0, 0)),
            pl.BlockSpec((1, 1), lambda i: (0, 0)),
        ],
        out_specs=pl.BlockSpec((_G, 1), lambda i: (0, 0)),
        out_shape=jax.ShapeDtypeStruct((_G, 1), jnp.float32),
        scratch_shapes=[
            pltpu.VMEM((_G, 128), jnp.float32),
            pltpu.VMEM((_G, 1), jnp.float32),
        ],
    )(dinv, p, v, b, batp, wfc, bfc)


def kernel(x, edge_index, batch, W1, b1, W2, b2, W3, b3, Wfc, bfc):
    npad = _NP - _N
    dum = (jnp.arange(_EP - _E, dtype=jnp.int32) % npad) + _N
    srcp = jnp.concatenate([edge_index[0], dum]).reshape(_EP // _K, _K)
    dstp = jnp.concatenate([edge_index[1], dum]).reshape(_EP // _K, _K)
    xp = jnp.concatenate([x, jnp.zeros((npad, _DIN), jnp.float32)])
    batp = jnp.concatenate(
        [batch, jnp.full((npad,), _G, batch.dtype)]).reshape(_NP, 1)

    ones1 = jnp.ones((_K,), jnp.float32)
    z1 = jnp.zeros((_RPT,), jnp.float32)
    z128 = jnp.zeros((_RPT, 128), jnp.float32)

    W3p = jnp.concatenate([W3, jnp.zeros((_DH, 64), jnp.float32)], axis=1)
    b3p = jnp.concatenate([b3, jnp.zeros((64,), jnp.float32)]).reshape(1, -1)
    Wfcp = jnp.concatenate([Wfc, jnp.zeros((64, 1), jnp.float32)], axis=0)

    degp = _sc_deg(dstp, ones1, z1)
    dinv, v1 = _tc0(degp, xp, W1)
    p1 = _spmm128(v1, srcp, dstp, z128)
    v2 = _tc_mid(dinv, p1, v1, b1.reshape(1, -1), W2, 128, 128)
    p2 = _spmm128(v2, srcp, dstp, z128)
    v3 = _tc_mid(dinv, p2, v2, b2.reshape(1, -1), W3p, 128, 128)
    p3 = _spmm128(v3, srcp, dstp, z128)
    out = _tc_final(dinv, p3, v3, b3p, batp, Wfcp, bfc.reshape(1, -1))
    return out.reshape(_G)
